# Initial kernel scaffold; baseline (speedup 1.0000x reference)
import jax
import jax.numpy as jnp
from jax.experimental import pallas as pl


def kernel(x, edge_index, W, att_src, att_dst, bias, bn_gamma, bn_beta):
    raise NotImplementedError("write your pallas kernel here")


# speedup vs baseline: 35.4471x; 35.4471x over previous
import functools

import jax
import jax.numpy as jnp
from jax import lax
from jax.experimental import pallas as pl
from jax.experimental.pallas import tpu as pltpu
from jax.experimental.pallas import tpu_sc as plsc

NEG_SLOPE = 0.2
BN_EPS = 1e-5

NC = 2
NS = 16
LANES = 16
BLK = 80
WPAD = 8


def _proj_body(nheads, cph, x_ref, w_ref, aw_s_ref, aw_d_ref,
               h2_ref, a8_ref):
    h = jnp.dot(x_ref[...], w_ref[...], preferred_element_type=jnp.float32)
    ho = h.shape[1]
    h2_ref[0] = h[:, :ho // 2]
    h2_ref[1] = h[:, ho // 2:]
    cols = []
    for aw_ref in (aw_s_ref, aw_d_ref):
        for hd in range(nheads):
            blk = h[:, hd * cph:(hd + 1) * cph]
            cols.append(jnp.sum(blk * aw_ref[pl.ds(hd, 1), :], axis=1,
                                keepdims=True))
    a8_ref[...] = jnp.concatenate(cols, axis=1)


def _proj(x, W, att_src, att_dst):
    n = x.shape[0]
    nheads, cph = att_src.shape
    ho = W.shape[1]
    bn = 1000
    return pl.pallas_call(
        functools.partial(_proj_body, nheads, cph),
        grid=(n // bn,),
        in_specs=[
            pl.BlockSpec((bn, x.shape[1]), lambda i: (i, 0)),
            pl.BlockSpec(W.shape, lambda i: (0, 0)),
            pl.BlockSpec(att_src.shape, lambda i: (0, 0)),
            pl.BlockSpec(att_dst.shape, lambda i: (0, 0)),
        ],
        out_specs=[
            pl.BlockSpec((NC, bn, ho // NC), lambda i: (0, i, 0)),
            pl.BlockSpec((bn, 2 * nheads), lambda i: (i, 0)),
        ],
        out_shape=[
            jax.ShapeDtypeStruct((NC, n, ho // NC), jnp.float32),
            jax.ShapeDtypeStruct((n, 2 * nheads), jnp.float32),
        ],
    )(x, W, att_src, att_dst)


def _sc_edges(h2, a8, eidx, shift16):
    _, n, cpc = h2.shape
    nheads = a8.shape[1] // 2
    hpc = nheads // NC
    cph = cpc // hpc
    e = eidx.shape[1]
    ept = e // NS
    nchunks = ept // BLK
    assert e % NS == 0 and ept % BLK == 0
    rpt = (n // NS + 7) // 8 * 8
    lrpt = n - (NS - 1) * rpt
    assert lrpt > 0 and lrpt % 8 == 0 and n % 8 == 0

    mesh = plsc.VectorSubcoreMesh(core_axis_name="c", subcore_axis_name="s")

    @functools.partial(
        pl.kernel,
        compiler_params=pltpu.CompilerParams(
            use_tc_tiling_on_sc=False, needs_layout_passes=False),
        out_type=[
            jax.ShapeDtypeStruct((NC, n, cpc), jnp.float32),
            jax.ShapeDtypeStruct((NC, n, WPAD), jnp.float32),
        ],
        mesh=mesh,
        scratch_types=[
            pltpu.VMEM((n, 2 * nheads), jnp.float32),
            pltpu.VMEM((2, BLK), jnp.int32),
            pltpu.VMEM((BLK, cpc), jnp.float32),
            pltpu.VMEM((BLK, WPAD), jnp.float32),
            pltpu.VMEM((LANES,), jnp.float32),
            pltpu.VMEM_SHARED((n, cpc), jnp.float32),
            pltpu.VMEM_SHARED((n, WPAD), jnp.float32),
            pltpu.SemaphoreType.DMA,
        ],
    )
    def k(h2_hbm, a8_hbm, ei_hbm, sh_hbm, zacc_hbm, zden_hbm,
          acc_out, den_out, a8_t, idx_v, hrow_v, w_v, sh_v,
          acc_sh, den_sh, sem):
        c = lax.axis_index("c")
        s = lax.axis_index("s")
        pltpu.sync_copy(a8_hbm, a8_t)
        pltpu.sync_copy(sh_hbm, sh_v)
        iota = lax.iota(jnp.int32, LANES)

        def zbody(z, cz):
            plsc.store_scatter(w_v, [2 * z + (iota >> 3), iota & 7],
                               jnp.zeros((LANES,), jnp.float32))
            return cz

        lax.fori_loop(0, BLK * WPAD // LANES, zbody, 0)
        rs = s * rpt

        @pl.when(s < NS - 1)
        def _():
            pltpu.sync_copy(zacc_hbm.at[pl.ds(rs, rpt)],
                            acc_sh.at[pl.ds(rs, rpt)])
            pltpu.sync_copy(zden_hbm.at[pl.ds(rs, rpt)],
                            den_sh.at[pl.ds(rs, rpt)])

        @pl.when(s == NS - 1)
        def _():
            pltpu.sync_copy(zacc_hbm.at[pl.ds(rs, lrpt)],
                            acc_sh.at[pl.ds(rs, lrpt)])
            pltpu.sync_copy(zden_hbm.at[pl.ds(rs, lrpt)],
                            den_sh.at[pl.ds(rs, lrpt)])

        plsc.subcore_barrier()

        oct_ = iota >> 1
        hp2 = iota & 1
        ghead = hp2 + hpc * c
        shv = plsc.load_gather(sh_v, [ghead])
        ebase = s * ept

        zero16 = iota * 0
        one16 = zero16 + 1

        def chunk_body(kk, carry):
            base = ebase + kk * BLK
            pltpu.sync_copy(ei_hbm.at[:, pl.ds(base, BLK)], idx_v)
            pltpu.async_copy(h2_hbm.at[c].at[idx_v.at[0]], hrow_v, sem).wait()

            def wbody(j, cw):
                erow = oct_ + 8 * j
                srcrep = plsc.load_gather(idx_v, [zero16, erow])
                dstrep = plsc.load_gather(idx_v, [one16, erow])
                asv = plsc.load_gather(a8_t, [srcrep, ghead])
                adv = plsc.load_gather(a8_t, [dstrep, ghead + nheads])
                al = asv + adv
                al = jnp.where(al >= 0.0, al, al * NEG_SLOPE)
                wv = jnp.exp(al - shv)
                plsc.store_scatter(w_v, [erow, hp2], wv)
                return cw

            lax.fori_loop(0, BLK // 8, wbody, 0, unroll=2)

            def ebody(ei, ce):
                for hd in range(hpc):
                    wsp = plsc.load_gather(
                        w_v, [jnp.full((LANES,), ei, jnp.int32),
                              jnp.full((LANES,), hd, jnp.int32)])
                    for q in range(cph // LANES):
                        col = hd * cph + q * LANES
                        hrow_v[ei, pl.ds(col, LANES)] = (
                            hrow_v[ei, pl.ds(col, LANES)] * wsp)
                return ce

            lax.fori_loop(0, BLK, ebody, 0, unroll=4)

            pltpu.sync_copy(hrow_v, acc_sh.at[idx_v.at[1]], add=True)
            pltpu.sync_copy(w_v, den_sh.at[idx_v.at[1]], add=True)
            return carry

        lax.fori_loop(0, nchunks, chunk_body, 0)
        plsc.subcore_barrier()

        @pl.when(s < NS - 1)
        def _():
            pltpu.sync_copy(acc_sh.at[pl.ds(rs, rpt)],
                            acc_out.at[c, pl.ds(rs, rpt)])
            pltpu.sync_copy(den_sh.at[pl.ds(rs, rpt)],
                            den_out.at[c, pl.ds(rs, rpt)])

        @pl.when(s == NS - 1)
        def _():
            pltpu.sync_copy(acc_sh.at[pl.ds(rs, lrpt)],
                            acc_out.at[c, pl.ds(rs, lrpt)])
            pltpu.sync_copy(den_sh.at[pl.ds(rs, lrpt)],
                            den_out.at[c, pl.ds(rs, lrpt)])

    zacc = jnp.zeros((n, cpc), jnp.float32)
    zden = jnp.zeros((n, WPAD), jnp.float32)
    return k(h2, a8, eidx, shift16, zacc, zden)


def _ep1_body(nheads, cph, acc_ref, den_ref, h_ref, a8_ref, sh_ref,
              bias_ref, o_ref, stats_ref):
    i = pl.program_id(0)
    bn = h_ref.shape[0]
    al = a8_ref[:, :nheads] + a8_ref[:, nheads:]
    al = jnp.where(al >= 0.0, al, al * NEG_SLOPE)
    ws = jnp.exp(al - sh_ref[...])
    dent = den_ref[...] + ws
    wrep = jnp.concatenate(
        [jnp.broadcast_to(ws[:, hd:hd + 1], (bn, cph)) for hd in range(nheads)],
        axis=1)
    drep = jnp.concatenate(
        [jnp.broadcast_to(dent[:, hd:hd + 1], (bn, cph))
         for hd in range(nheads)], axis=1)
    acct = acc_ref[...] + wrep * h_ref[...]
    o = acct / (drep + 1e-16) + bias_ref[...]
    o_ref[...] = o
    st = jnp.concatenate([jnp.sum(o, axis=0, keepdims=True),
                          jnp.sum(o * o, axis=0, keepdims=True)], axis=0)

    @pl.when(i == 0)
    def _():
        stats_ref[...] = jnp.zeros_like(stats_ref)

    stats_ref[...] += st


def _ep1(acc, den4, h, a8, shift, bias):
    n, ho = h.shape
    nheads = a8.shape[1] // 2
    cph = ho // nheads
    bn = 1000
    full = lambda a: pl.BlockSpec(a.shape, lambda i: tuple(0 for _ in a.shape))
    row = lambda w: pl.BlockSpec((bn, w), lambda i: (i, 0))
    return pl.pallas_call(
        functools.partial(_ep1_body, nheads, cph),
        grid=(n // bn,),
        in_specs=[row(ho), row(nheads), row(ho), row(2 * nheads),
                  full(shift), full(bias)],
        out_specs=[row(ho), pl.BlockSpec((2, ho), lambda i: (0, 0))],
        out_shape=[jax.ShapeDtypeStruct((n, ho), jnp.float32),
                   jax.ShapeDtypeStruct((2, ho), jnp.float32)],
    )(acc, den4, h, a8, shift, bias)


def _ep2_body(o_ref, mean_ref, var_ref, g_ref, b_ref, out_ref):
    xn = (o_ref[...] - mean_ref[...]) * lax.rsqrt(var_ref[...] + BN_EPS)
    xn = xn * g_ref[...] + b_ref[...]
    out_ref[...] = jnp.where(xn > 0.0, xn, jnp.exp(xn) - 1.0)


def _ep2(o, mean, var, gamma, beta):
    n, ho = o.shape
    bn = 1000
    full = lambda a: pl.BlockSpec(a.shape, lambda i: tuple(0 for _ in a.shape))
    return pl.pallas_call(
        _ep2_body,
        grid=(n // bn,),
        in_specs=[pl.BlockSpec((bn, ho), lambda i: (i, 0)),
                  full(mean), full(var), full(gamma), full(beta)],
        out_specs=pl.BlockSpec((bn, ho), lambda i: (i, 0)),
        out_shape=jax.ShapeDtypeStruct((n, ho), jnp.float32),
    )(o, mean, var, gamma, beta)


def kernel(x, edge_index, W, att_src, att_dst, bias, bn_gamma, bn_beta):
    n = x.shape[0]
    ho = W.shape[1]
    nheads = att_src.shape[0]
    hpc = nheads // NC

    h2, a8 = _proj(x, W, att_src, att_dst)

    m = jnp.max(a8[:, :nheads], axis=0) + jnp.max(a8[:, nheads:], axis=0)
    shift = jnp.where(m >= 0.0, m, m * NEG_SLOPE)
    shift16 = jnp.tile(shift, LANES // nheads)

    acc, den = _sc_edges(h2, a8, edge_index, shift16)

    h_full = jnp.concatenate([h2[0], h2[1]], axis=1)
    acc_full = jnp.concatenate([acc[0], acc[1]], axis=1)
    den4 = jnp.concatenate([den[0, :, :hpc], den[1, :, :hpc]], axis=1)

    o, stats = _ep1(acc_full, den4, h_full, a8,
                    shift.reshape(1, nheads), bias.reshape(1, ho))
    mean = (stats[0] / n).reshape(1, ho)
    var = (stats[1] / n).reshape(1, ho) - mean * mean
    return _ep2(o, mean, var, bn_gamma.reshape(1, ho), bn_beta.reshape(1, ho))
